# baseline (speedup 1.0000x reference)
import jax
import jax.numpy as jnp
from jax.experimental import pallas as pl
from jax.experimental.pallas import tpu as pltpu

_B, _N, _W, _R = 8, 2048, 64, 4
_TI = 512
_NI = _N // _TI
_TA = 256
_F32 = jnp.float32


def _body(mem_ref, u_ref, ut_ref, wkey_ref, wstr_ref, ag_ref, wg_ref,
          wvec_ref, evec_ref, rkeys_ref, rstr_ref,
          L_ref, prec_ref, prec_all_ref, rw_ref, rws_ref, rm_ref, fg_ref,
          Lout_ref, memnew_ref, rwout_ref, rv_ref, uout_ref, pout_ref,
          memnew_s, cw_s, ww_s, wwc_s, fw_s, bw_s, wwall_s):
    b = pl.program_id(0)
    i = pl.program_id(1)

    @pl.when(i == 0)
    def _():
        mem = mem_ref[0]
        u = u_ref[0]
        u_col = ut_ref[0]

        lu_col = jnp.log(jnp.maximum(u_col, 1e-37))
        iota_j = jax.lax.broadcasted_iota(jnp.int32, (_TA, _N), 1)
        chunks = []
        for c in range(_N // _TA):
            ui_col = u_col[c * _TA:(c + 1) * _TA]
            ii_col = (jax.lax.broadcasted_iota(jnp.int32, (_TA, 1), 0)
                      + (c * _TA))
            lt = u < ui_col
            eq = u == ui_col
            mask = jnp.where(
                jnp.logical_or(lt, jnp.logical_and(eq, iota_j < ii_col)),
                1.0, 0.0)
            s = jax.lax.dot_general(mask, lu_col, (((1,), (0,)), ((), ())),
                                    preferred_element_type=_F32)
            chunks.append(s)
        s_row = jnp.transpose(jnp.concatenate(chunks, axis=0))
        alloc = (1.0 - u) * jnp.exp(s_row)

        wk = wkey_ref[0]
        ones_w = jnp.ones((1, _W), _F32)
        dot_col = jax.lax.dot_general(mem, wk, (((1,), (1,)), ((), ())),
                                      preferred_element_type=_F32)
        mn2_col = jax.lax.dot_general(mem * mem, ones_w,
                                      (((1,), (1,)), ((), ())),
                                      preferred_element_type=_F32)
        kn = jnp.sqrt(jnp.sum(wk * wk, axis=1, keepdims=True))
        dot = jnp.transpose(dot_col)
        mn2_row = jnp.transpose(mn2_col)
        sim = dot / jnp.maximum(kn * jnp.sqrt(mn2_row), 1e-8)
        e = jnp.exp(sim - jnp.max(sim, axis=1, keepdims=True))
        wc = e / jnp.sum(e, axis=1, keepdims=True) * wstr_ref[0]

        ww = (ag_ref[0] * (alloc - wc) + wc) * wg_ref[0]
        ww_s[...] = ww
        ww_col = jnp.transpose(ww)
        wwc_s[...] = ww_col
        wwall_s[pl.ds(b, 1), :] = ww

        memnew = mem * (1.0 - ww_col * evec_ref[0]) + ww_col * wvec_ref[0]
        memnew_ref[0] = memnew
        memnew_s[...] = memnew

        rk = rkeys_ref[0]
        dot_r = jax.lax.dot_general(rk, memnew, (((1,), (1,)), ((), ())),
                                    preferred_element_type=_F32)
        mn2b = jax.lax.dot_general(ones_w, memnew * memnew,
                                   (((1,), (1,)), ((), ())),
                                   preferred_element_type=_F32)
        rn = jnp.sqrt(jnp.sum(rk * rk, axis=1, keepdims=True))
        sim_r = dot_r / jnp.maximum(rn * jnp.sqrt(mn2b), 1e-8)
        e_r = jnp.exp(sim_r - jnp.max(sim_r, axis=1, keepdims=True))
        cw_s[...] = (e_r / jnp.sum(e_r, axis=1, keepdims=True)
                     * rstr_ref[0])

        bw_s[...] = jnp.zeros((_R, _N), _F32)

    ww = ww_s[...]
    prec = prec_ref[0]
    idx = pl.multiple_of(i * _TI, _TI)
    wwi = wwc_s[pl.ds(idx, _TI), :]
    L = L_ref[0]

    Lnew = L * (1.0 - wwi - ww) + wwi * prec
    Lout_ref[0] = Lnew

    rw = rw_ref[0]
    fw_strip = jax.lax.dot_general(rw, Lnew, (((1,), (1,)), ((), ())),
                                   preferred_element_type=_F32)
    fw_s[:, pl.ds(idx, _TI)] = fw_strip

    bw_s[...] += jax.lax.dot_general(rws_ref[0], Lnew, (((1,), (0,)), ((), ())),
                                     preferred_element_type=_F32)

    @pl.when(i == _NI - 1)
    def _():
        rm = rm_ref[0]
        rw_new = (fw_s[...] * rm[:, 0:1] + bw_s[...] * rm[:, 1:2]
                  + cw_s[...] * rm[:, 2:3])
        rwout_ref[0] = rw_new
        rv_ref[0] = jax.lax.dot_general(rw_new, memnew_s[...],
                                        (((1,), (0,)), ((), ())),
                                        preferred_element_type=_F32)
        prodw = (rw_new[0:1] * rw_new[1:2]) * (rw_new[2:3] * rw_new[3:4])
        ret = 1.0 - fg_ref[0] * prodw
        uold = u_ref[0]
        uout_ref[0] = (uold + ww - uold * ww) * ret

    @pl.when(jnp.logical_and(b == _B - 1, i == _NI - 1))
    def _():
        wsum = jnp.sum(wwall_s[...])
        prec_all = prec_all_ref[...]
        pout_ref[:, 0, :] = (1.0 - wsum) * prec_all[:, 0, :] + wwall_s[...]


def kernel(memory, usage, read_weights, temporal_link, precedence, write_key,
           write_strength, allocation_gate, write_gate, write_vector,
           erase_vector, read_keys, read_strength, read_modes, free_gates):
    f32 = _F32
    usage_t = jnp.transpose(usage, (0, 2, 1))

    (L_new, mem_new, rw_new, read_val, usage_new, prec_new) = pl.pallas_call(
        _body,
        grid=(_B, _NI),
        in_specs=[
            pl.BlockSpec((1, _N, _W), lambda b, i: (b, 0, 0)),
            pl.BlockSpec((1, 1, _N), lambda b, i: (b, 0, 0)),
            pl.BlockSpec((1, _N, 1), lambda b, i: (b, 0, 0)),
            pl.BlockSpec((1, 1, _W), lambda b, i: (b, 0, 0)),
            pl.BlockSpec((1, 1, 1), lambda b, i: (b, 0, 0)),
            pl.BlockSpec((1, 1, 1), lambda b, i: (b, 0, 0)),
            pl.BlockSpec((1, 1, 1), lambda b, i: (b, 0, 0)),
            pl.BlockSpec((1, 1, _W), lambda b, i: (b, 0, 0)),
            pl.BlockSpec((1, 1, _W), lambda b, i: (b, 0, 0)),
            pl.BlockSpec((1, _R, _W), lambda b, i: (b, 0, 0)),
            pl.BlockSpec((1, _R, 1), lambda b, i: (b, 0, 0)),
            pl.BlockSpec((1, _TI, _N), lambda b, i: (b, i, 0)),
            pl.BlockSpec((1, 1, _N), lambda b, i: (b, 0, 0)),
            pl.BlockSpec((_B, 1, _N), lambda b, i: (0, 0, 0)),
            pl.BlockSpec((1, _R, _N), lambda b, i: (b, 0, 0)),
            pl.BlockSpec((1, _R, _TI), lambda b, i: (b, 0, i)),
            pl.BlockSpec((1, _R, 3), lambda b, i: (b, 0, 0)),
            pl.BlockSpec((1, 1, _N), lambda b, i: (b, 0, 0)),
        ],
        out_specs=[
            pl.BlockSpec((1, _TI, _N), lambda b, i: (b, i, 0)),
            pl.BlockSpec((1, _N, _W), lambda b, i: (b, 0, 0)),
            pl.BlockSpec((1, _R, _N), lambda b, i: (b, 0, 0)),
            pl.BlockSpec((1, _R, _W), lambda b, i: (b, 0, 0)),
            pl.BlockSpec((1, 1, _N), lambda b, i: (b, 0, 0)),
            pl.BlockSpec((_B, 1, _N), lambda b, i: (0, 0, 0)),
        ],
        out_shape=[
            jax.ShapeDtypeStruct((_B, _N, _N), f32),
            jax.ShapeDtypeStruct((_B, _N, _W), f32),
            jax.ShapeDtypeStruct((_B, _R, _N), f32),
            jax.ShapeDtypeStruct((_B, _R, _W), f32),
            jax.ShapeDtypeStruct((_B, 1, _N), f32),
            jax.ShapeDtypeStruct((_B, 1, _N), f32),
        ],
        scratch_shapes=[
            pltpu.VMEM((_N, _W), f32),
            pltpu.VMEM((_R, _N), f32),
            pltpu.VMEM((1, _N), f32),
            pltpu.VMEM((_N, 1), f32),
            pltpu.VMEM((_R, _N), f32),
            pltpu.VMEM((_R, _N), f32),
            pltpu.VMEM((_B, _N), f32),
        ],
    )(memory, usage, usage_t, write_key, write_strength, allocation_gate,
      write_gate, write_vector, erase_vector, read_keys, read_strength,
      temporal_link, precedence, precedence, read_weights, read_weights,
      read_modes, free_gates)

    return (read_val, mem_new, usage_new, rw_new, L_new, prec_new)


# speedup vs baseline: 1.0149x; 1.0149x over previous
import jax
import jax.numpy as jnp
from jax.experimental import pallas as pl
from jax.experimental.pallas import tpu as pltpu

_B, _N, _W, _R = 8, 2048, 64, 4
_TI = 512
_NI = _N // _TI
_TA = 256
_F32 = jnp.float32


def _addr_body(mem_ref, u_ref, ut_ref, wkey_ref, wstr_ref, ag_ref, wg_ref,
               wvec_ref, evec_ref, rkeys_ref, rstr_ref,
               ww_ref, wwc_ref, memnew_ref, cw_ref, wsum_ref):
    b = pl.program_id(0)
    mem = mem_ref[0]
    u = u_ref[0]
    u_col = ut_ref[0]

    lu_col = jnp.log(jnp.maximum(u_col, 1e-37))
    iota_j = jax.lax.broadcasted_iota(jnp.int32, (_TA, _N), 1)
    chunks = []
    for c in range(_N // _TA):
        ui_col = u_col[c * _TA:(c + 1) * _TA]
        ii_col = jax.lax.broadcasted_iota(jnp.int32, (_TA, 1), 0) + (c * _TA)
        lt = u < ui_col
        eq = u == ui_col
        mask = jnp.where(jnp.logical_or(lt, jnp.logical_and(eq, iota_j < ii_col)),
                         1.0, 0.0)
        s = jax.lax.dot_general(mask, lu_col, (((1,), (0,)), ((), ())),
                                preferred_element_type=_F32)
        chunks.append(s)
    s_row = jnp.transpose(jnp.concatenate(chunks, axis=0))
    alloc = (1.0 - u) * jnp.exp(s_row)

    wk = wkey_ref[0]
    ones_w = jnp.ones((1, _W), _F32)
    dot_col = jax.lax.dot_general(mem, wk, (((1,), (1,)), ((), ())),
                                  preferred_element_type=_F32)
    mn2_col = jax.lax.dot_general(mem * mem, ones_w, (((1,), (1,)), ((), ())),
                                  preferred_element_type=_F32)
    kn = jnp.sqrt(jnp.sum(wk * wk, axis=1, keepdims=True))
    dot = jnp.transpose(dot_col)
    mn2_row = jnp.transpose(mn2_col)
    sim = dot / jnp.maximum(kn * jnp.sqrt(mn2_row), 1e-8)
    e = jnp.exp(sim - jnp.max(sim, axis=1, keepdims=True))
    wc = e / jnp.sum(e, axis=1, keepdims=True) * wstr_ref[0]

    ww = (ag_ref[0] * (alloc - wc) + wc) * wg_ref[0]
    ww_ref[0] = ww
    ww_col = jnp.transpose(ww)
    wwc_ref[0] = ww_col

    @pl.when(b == 0)
    def _():
        wsum_ref[...] = jnp.zeros((1, 1), _F32)
    wsum_ref[...] += jnp.sum(ww, axis=(0, 1), keepdims=True)

    memnew = mem * (1.0 - ww_col * evec_ref[0]) + ww_col * wvec_ref[0]
    memnew_ref[0] = memnew

    rk = rkeys_ref[0]
    dot_r = jax.lax.dot_general(rk, memnew, (((1,), (1,)), ((), ())),
                                preferred_element_type=_F32)
    mn2b = jax.lax.dot_general(ones_w, memnew * memnew,
                               (((1,), (1,)), ((), ())),
                               preferred_element_type=_F32)
    rn = jnp.sqrt(jnp.sum(rk * rk, axis=1, keepdims=True))
    sim_r = dot_r / jnp.maximum(rn * jnp.sqrt(mn2b), 1e-8)
    e_r = jnp.exp(sim_r - jnp.max(sim_r, axis=1, keepdims=True))
    cw_ref[0] = e_r / jnp.sum(e_r, axis=1, keepdims=True) * rstr_ref[0]


def _link_body(L_ref, ww_ref, wwc_ref, prec_ref, rw_ref, rws_ref, cw_ref,
               mem_ref, fg_ref, u_ref, rm_ref, wsum_ref,
               Lout_ref, rwout_ref, rv_ref, uout_ref, pout_ref,
               fw_s, bw_s):
    i = pl.program_id(1)
    L = L_ref[0]
    ww = ww_ref[0]
    prec = prec_ref[0]
    wwi = wwc_ref[0]

    Lnew = L * (1.0 - wwi - ww) + wwi * prec
    Lout_ref[0] = Lnew

    rw = rw_ref[0]
    fw_strip = jax.lax.dot_general(rw, Lnew, (((1,), (1,)), ((), ())),
                                   preferred_element_type=_F32)
    idx = pl.multiple_of(i * _TI, _TI)
    fw_s[:, pl.ds(idx, _TI)] = fw_strip

    bw_c = jax.lax.dot_general(rws_ref[0], Lnew, (((1,), (0,)), ((), ())),
                               preferred_element_type=_F32)

    @pl.when(i == 0)
    def _():
        bw_s[...] = jnp.zeros((_R, _N), _F32)
    bw_s[...] += bw_c

    @pl.when(i == _NI - 1)
    def _():
        rm = rm_ref[0]
        rw_new = (fw_s[...] * rm[:, 0:1] + bw_s[...] * rm[:, 1:2]
                  + cw_ref[0] * rm[:, 2:3])
        rwout_ref[0] = rw_new
        rv_ref[0] = jax.lax.dot_general(rw_new, mem_ref[0],
                                        (((1,), (0,)), ((), ())),
                                        preferred_element_type=_F32)
        prodw = (rw_new[0:1] * rw_new[1:2]) * (rw_new[2:3] * rw_new[3:4])
        ret = 1.0 - fg_ref[0] * prodw
        uold = u_ref[0]
        uout_ref[0] = (uold + ww - uold * ww) * ret
        pout_ref[0] = (1.0 - wsum_ref[0]) * prec + ww


def kernel(memory, usage, read_weights, temporal_link, precedence, write_key,
           write_strength, allocation_gate, write_gate, write_vector,
           erase_vector, read_keys, read_strength, read_modes, free_gates):
    f32 = _F32
    usage_t = jnp.transpose(usage, (0, 2, 1))

    ww, ww_col, mem_new, cw, wsum = pl.pallas_call(
        _addr_body,
        grid=(_B,),
        in_specs=[
            pl.BlockSpec((1, _N, _W), lambda b: (b, 0, 0)),
            pl.BlockSpec((1, 1, _N), lambda b: (b, 0, 0)),
            pl.BlockSpec((1, _N, 1), lambda b: (b, 0, 0)),
            pl.BlockSpec((1, 1, _W), lambda b: (b, 0, 0)),
            pl.BlockSpec((1, 1, 1), lambda b: (b, 0, 0)),
            pl.BlockSpec((1, 1, 1), lambda b: (b, 0, 0)),
            pl.BlockSpec((1, 1, 1), lambda b: (b, 0, 0)),
            pl.BlockSpec((1, 1, _W), lambda b: (b, 0, 0)),
            pl.BlockSpec((1, 1, _W), lambda b: (b, 0, 0)),
            pl.BlockSpec((1, _R, _W), lambda b: (b, 0, 0)),
            pl.BlockSpec((1, _R, 1), lambda b: (b, 0, 0)),
        ],
        out_specs=[
            pl.BlockSpec((1, 1, _N), lambda b: (b, 0, 0)),
            pl.BlockSpec((1, _N, 1), lambda b: (b, 0, 0)),
            pl.BlockSpec((1, _N, _W), lambda b: (b, 0, 0)),
            pl.BlockSpec((1, _R, _N), lambda b: (b, 0, 0)),
            pl.BlockSpec((1, 1), lambda b: (0, 0)),
        ],
        out_shape=[
            jax.ShapeDtypeStruct((_B, 1, _N), f32),
            jax.ShapeDtypeStruct((_B, _N, 1), f32),
            jax.ShapeDtypeStruct((_B, _N, _W), f32),
            jax.ShapeDtypeStruct((_B, _R, _N), f32),
            jax.ShapeDtypeStruct((1, 1), f32),
        ],
    )(memory, usage, usage_t, write_key, write_strength, allocation_gate,
      write_gate, write_vector, erase_vector, read_keys, read_strength)

    L_new, rw_new, read_val, usage_new, prec_new = pl.pallas_call(
        _link_body,
        grid=(_B, _NI),
        in_specs=[
            pl.BlockSpec((1, _TI, _N), lambda b, i: (b, i, 0)),
            pl.BlockSpec((1, 1, _N), lambda b, i: (b, 0, 0)),
            pl.BlockSpec((1, _TI, 1), lambda b, i: (b, i, 0)),
            pl.BlockSpec((1, 1, _N), lambda b, i: (b, 0, 0)),
            pl.BlockSpec((1, _R, _N), lambda b, i: (b, 0, 0)),
            pl.BlockSpec((1, _R, _TI), lambda b, i: (b, 0, i)),
            pl.BlockSpec((1, _R, _N), lambda b, i: (b, 0, 0)),
            pl.BlockSpec((1, _N, _W), lambda b, i: (b, 0, 0)),
            pl.BlockSpec((1, 1, _N), lambda b, i: (b, 0, 0)),
            pl.BlockSpec((1, 1, _N), lambda b, i: (b, 0, 0)),
            pl.BlockSpec((1, _R, 3), lambda b, i: (b, 0, 0)),
            pl.BlockSpec((1, 1), lambda b, i: (0, 0)),
        ],
        out_specs=[
            pl.BlockSpec((1, _TI, _N), lambda b, i: (b, i, 0)),
            pl.BlockSpec((1, _R, _N), lambda b, i: (b, 0, 0)),
            pl.BlockSpec((1, _R, _W), lambda b, i: (b, 0, 0)),
            pl.BlockSpec((1, 1, _N), lambda b, i: (b, 0, 0)),
            pl.BlockSpec((1, 1, _N), lambda b, i: (b, 0, 0)),
        ],
        out_shape=[
            jax.ShapeDtypeStruct((_B, _N, _N), f32),
            jax.ShapeDtypeStruct((_B, _R, _N), f32),
            jax.ShapeDtypeStruct((_B, _R, _W), f32),
            jax.ShapeDtypeStruct((_B, 1, _N), f32),
            jax.ShapeDtypeStruct((_B, 1, _N), f32),
        ],
        scratch_shapes=[
            pltpu.VMEM((_R, _N), f32),
            pltpu.VMEM((_R, _N), f32),
        ],
    )(temporal_link, ww, ww_col, precedence, read_weights, read_weights, cw,
      mem_new, free_gates, usage, read_modes, wsum)

    return (read_val, mem_new, usage_new, rw_new, L_new, prec_new)


# speedup vs baseline: 1.0889x; 1.0729x over previous
import jax
import jax.numpy as jnp
from jax.experimental import pallas as pl
from jax.experimental.pallas import tpu as pltpu

_B, _N, _W, _R = 8, 2048, 64, 4
_TI = 512
_NI = _N // _TI
_TA = 256
_F32 = jnp.float32


def _body(mem_ref, u_ref, ut_ref, wkey_ref, wstr_ref, ag_ref, wg_ref,
          wvec_ref, evec_ref, rkeys_ref, rstr_ref,
          L_ref, rws_ref,
          prec_ref, rw_ref, rm_ref, fg_ref, u2_ref, mem2_ref, evec2_ref,
          wvec2_ref, prec_all_ref,
          Lout_ref, memnew_ref, rwout_ref, rv_ref, uout_ref, pout_ref,
          wwall_s, wwcol_s, cwall_s, fw_s, bw_s):
    b = pl.program_id(0)
    i = pl.program_id(1)
    ks = b * _NI + i

    @pl.when(ks < _B)
    def _():
        mem = mem_ref[0]
        u = u_ref[0]
        u_col = ut_ref[0]

        lu_col = jnp.log(jnp.maximum(u_col, 1e-37))
        iota_j = jax.lax.broadcasted_iota(jnp.int32, (_TA, _N), 1)
        chunks = []
        for c in range(_N // _TA):
            ui_col = u_col[c * _TA:(c + 1) * _TA]
            ii_col = (jax.lax.broadcasted_iota(jnp.int32, (_TA, 1), 0)
                      + (c * _TA))
            lt = u < ui_col
            eq = u == ui_col
            mask = jnp.where(
                jnp.logical_or(lt, jnp.logical_and(eq, iota_j < ii_col)),
                1.0, 0.0)
            s = jax.lax.dot_general(mask, lu_col, (((1,), (0,)), ((), ())),
                                    preferred_element_type=_F32)
            chunks.append(s)
        s_row = jnp.transpose(jnp.concatenate(chunks, axis=0))
        alloc = (1.0 - u) * jnp.exp(s_row)

        wk = wkey_ref[0]
        ones_w = jnp.ones((1, _W), _F32)
        dot_col = jax.lax.dot_general(mem, wk, (((1,), (1,)), ((), ())),
                                      preferred_element_type=_F32)
        mn2_col = jax.lax.dot_general(mem * mem, ones_w,
                                      (((1,), (1,)), ((), ())),
                                      preferred_element_type=_F32)
        kn = jnp.sqrt(jnp.sum(wk * wk, axis=1, keepdims=True))
        dot = jnp.transpose(dot_col)
        mn2_row = jnp.transpose(mn2_col)
        sim = dot / jnp.maximum(kn * jnp.sqrt(mn2_row), 1e-8)
        e = jnp.exp(sim - jnp.max(sim, axis=1, keepdims=True))
        wc = e / jnp.sum(e, axis=1, keepdims=True) * wstr_ref[0]

        ww = (ag_ref[0] * (alloc - wc) + wc) * wg_ref[0]
        wwall_s[pl.ds(ks, 1), :] = ww
        ww_col = jnp.transpose(ww)

        memnew = mem * (1.0 - ww_col * evec_ref[0]) + ww_col * wvec_ref[0]
        memnew_ref[0] = memnew

        rk = rkeys_ref[0]
        dot_r = jax.lax.dot_general(rk, memnew, (((1,), (1,)), ((), ())),
                                    preferred_element_type=_F32)
        mn2b = jax.lax.dot_general(ones_w, memnew * memnew,
                                   (((1,), (1,)), ((), ())),
                                   preferred_element_type=_F32)
        rn = jnp.sqrt(jnp.sum(rk * rk, axis=1, keepdims=True))
        sim_r = dot_r / jnp.maximum(rn * jnp.sqrt(mn2b), 1e-8)
        e_r = jnp.exp(sim_r - jnp.max(sim_r, axis=1, keepdims=True))
        cwall_s[:, pl.ds(ks * _N, _N)] = (
            e_r / jnp.sum(e_r, axis=1, keepdims=True) * rstr_ref[0])

    @pl.when(i == 0)
    def _():
        wwcol_s[...] = jnp.transpose(wwall_s[pl.ds(b, 1), :])
        bw_s[...] = jnp.zeros((_R, _N), _F32)

    ww = wwall_s[pl.ds(b, 1), :]
    prec = prec_ref[0]
    idx = pl.multiple_of(i * _TI, _TI)
    wwi = wwcol_s[pl.ds(idx, _TI), :]
    L = L_ref[0]

    Lnew = L * (1.0 - wwi - ww) + wwi * prec
    Lout_ref[0] = Lnew

    rw = rw_ref[0]
    fw_strip = jax.lax.dot_general(rw, Lnew, (((1,), (1,)), ((), ())),
                                   preferred_element_type=_F32)
    fw_s[:, pl.ds(idx, _TI)] = fw_strip

    bw_s[...] += jax.lax.dot_general(rws_ref[0], Lnew, (((1,), (0,)), ((), ())),
                                     preferred_element_type=_F32)

    @pl.when(i == _NI - 1)
    def _():
        rm = rm_ref[0]
        cw = cwall_s[:, pl.ds(b * _N, _N)]
        rw_new = (fw_s[...] * rm[:, 0:1] + bw_s[...] * rm[:, 1:2]
                  + cw * rm[:, 2:3])
        rwout_ref[0] = rw_new

        mem = mem2_ref[0]
        ev = evec2_ref[0]
        wv = wvec2_ref[0]
        rww = rw_new * ww
        t1 = jax.lax.dot_general(rw_new, mem, (((1,), (0,)), ((), ())),
                                 preferred_element_type=_F32)
        t2 = jax.lax.dot_general(rww, mem, (((1,), (0,)), ((), ())),
                                 preferred_element_type=_F32)
        q = jnp.sum(rww, axis=1, keepdims=True)
        rv_ref[0] = t1 - t2 * ev + q * wv

        prodw = (rw_new[0:1] * rw_new[1:2]) * (rw_new[2:3] * rw_new[3:4])
        ret = 1.0 - fg_ref[0] * prodw
        uold = u2_ref[0]
        uout_ref[0] = (uold + ww - uold * ww) * ret

    @pl.when(jnp.logical_and(b == _B - 1, i == _NI - 1))
    def _():
        wsum = jnp.sum(wwall_s[...])
        prec_all = prec_all_ref[...]
        pout_ref[:, 0, :] = (1.0 - wsum) * prec_all[:, 0, :] + wwall_s[...]


def _kidx(b, i):
    return jnp.minimum(b * _NI + i, _B - 1)


def kernel(memory, usage, read_weights, temporal_link, precedence, write_key,
           write_strength, allocation_gate, write_gate, write_vector,
           erase_vector, read_keys, read_strength, read_modes, free_gates):
    f32 = _F32
    usage_t = jnp.transpose(usage, (0, 2, 1))

    (L_new, mem_new, rw_new, read_val, usage_new, prec_new) = pl.pallas_call(
        _body,
        grid=(_B, _NI),
        in_specs=[
            pl.BlockSpec((1, _N, _W), lambda b, i: (_kidx(b, i), 0, 0)),
            pl.BlockSpec((1, 1, _N), lambda b, i: (_kidx(b, i), 0, 0)),
            pl.BlockSpec((1, _N, 1), lambda b, i: (_kidx(b, i), 0, 0)),
            pl.BlockSpec((1, 1, _W), lambda b, i: (_kidx(b, i), 0, 0)),
            pl.BlockSpec((1, 1, 1), lambda b, i: (_kidx(b, i), 0, 0)),
            pl.BlockSpec((1, 1, 1), lambda b, i: (_kidx(b, i), 0, 0)),
            pl.BlockSpec((1, 1, 1), lambda b, i: (_kidx(b, i), 0, 0)),
            pl.BlockSpec((1, 1, _W), lambda b, i: (_kidx(b, i), 0, 0)),
            pl.BlockSpec((1, 1, _W), lambda b, i: (_kidx(b, i), 0, 0)),
            pl.BlockSpec((1, _R, _W), lambda b, i: (_kidx(b, i), 0, 0)),
            pl.BlockSpec((1, _R, 1), lambda b, i: (_kidx(b, i), 0, 0)),
            pl.BlockSpec((1, _TI, _N), lambda b, i: (b, i, 0)),
            pl.BlockSpec((1, _R, _TI), lambda b, i: (b, 0, i)),
            pl.BlockSpec((1, 1, _N), lambda b, i: (b, 0, 0)),
            pl.BlockSpec((1, _R, _N), lambda b, i: (b, 0, 0)),
            pl.BlockSpec((1, _R, 3), lambda b, i: (b, 0, 0)),
            pl.BlockSpec((1, 1, _N), lambda b, i: (b, 0, 0)),
            pl.BlockSpec((1, 1, _N), lambda b, i: (b, 0, 0)),
            pl.BlockSpec((1, _N, _W), lambda b, i: (b, 0, 0)),
            pl.BlockSpec((1, 1, _W), lambda b, i: (b, 0, 0)),
            pl.BlockSpec((1, 1, _W), lambda b, i: (b, 0, 0)),
            pl.BlockSpec((_B, 1, _N), lambda b, i: (0, 0, 0)),
        ],
        out_specs=[
            pl.BlockSpec((1, _TI, _N), lambda b, i: (b, i, 0)),
            pl.BlockSpec((1, _N, _W), lambda b, i: (_kidx(b, i), 0, 0)),
            pl.BlockSpec((1, _R, _N), lambda b, i: (b, 0, 0)),
            pl.BlockSpec((1, _R, _W), lambda b, i: (b, 0, 0)),
            pl.BlockSpec((1, 1, _N), lambda b, i: (b, 0, 0)),
            pl.BlockSpec((_B, 1, _N), lambda b, i: (0, 0, 0)),
        ],
        out_shape=[
            jax.ShapeDtypeStruct((_B, _N, _N), f32),
            jax.ShapeDtypeStruct((_B, _N, _W), f32),
            jax.ShapeDtypeStruct((_B, _R, _N), f32),
            jax.ShapeDtypeStruct((_B, _R, _W), f32),
            jax.ShapeDtypeStruct((_B, 1, _N), f32),
            jax.ShapeDtypeStruct((_B, 1, _N), f32),
        ],
        scratch_shapes=[
            pltpu.VMEM((_B, _N), f32),
            pltpu.VMEM((_N, 1), f32),
            pltpu.VMEM((_R, _B * _N), f32),
            pltpu.VMEM((_R, _N), f32),
            pltpu.VMEM((_R, _N), f32),
        ],
    )(memory, usage, usage_t, write_key, write_strength, allocation_gate,
      write_gate, write_vector, erase_vector, read_keys, read_strength,
      temporal_link, read_weights,
      precedence, read_weights, read_modes, free_gates, usage, memory,
      erase_vector, write_vector, precedence)

    return (read_val, mem_new, usage_new, rw_new, L_new, prec_new)


# speedup vs baseline: 1.1870x; 1.0901x over previous
import jax
import jax.numpy as jnp
from jax.experimental import pallas as pl
from jax.experimental.pallas import tpu as pltpu

_B, _N, _W, _R = 8, 2048, 64, 4
_TI = 1024
_NI = _N // _TI
_TA = 256
_F32 = jnp.float32


def _body(mem_ref, u_ref, ut_ref, wkey_ref, wstr_ref, ag_ref, wg_ref,
          wvec_ref, evec_ref, rkeys_ref, rstr_ref,
          L_ref, rws_ref,
          prec_ref, rw_ref, rm_ref, fg_ref, u2_ref, prec_all_ref,
          Lout_ref, memnew_ref, rwout_ref, rv_ref, uout_ref, pout_ref,
          wwall_s, wwcol_s, cwall_s, memall_s, fw_s, bw_s):
    b = pl.program_id(0)
    i = pl.program_id(1)
    ks = b * _NI + i

    @pl.when(ks < _B)
    def _():
        mem = mem_ref[0]
        u = u_ref[0]
        u_col = ut_ref[0]

        lu_col = jnp.log(jnp.maximum(u_col, 1e-37))
        iota_j = jax.lax.broadcasted_iota(jnp.int32, (_TA, _N), 1)
        chunks = []
        for c in range(_N // _TA):
            ui_col = u_col[c * _TA:(c + 1) * _TA]
            ii_col = (jax.lax.broadcasted_iota(jnp.int32, (_TA, 1), 0)
                      + (c * _TA))
            lt = u < ui_col
            eq = u == ui_col
            mask = jnp.where(
                jnp.logical_or(lt, jnp.logical_and(eq, iota_j < ii_col)),
                1.0, 0.0)
            s = jax.lax.dot_general(mask, lu_col, (((1,), (0,)), ((), ())),
                                    preferred_element_type=_F32)
            chunks.append(s)
        s_row = jnp.transpose(jnp.concatenate(chunks, axis=0))
        alloc = (1.0 - u) * jnp.exp(s_row)

        wk = wkey_ref[0]
        ones_w = jnp.ones((1, _W), _F32)
        dot_col = jax.lax.dot_general(mem, wk, (((1,), (1,)), ((), ())),
                                      preferred_element_type=_F32)
        mn2_col = jax.lax.dot_general(mem * mem, ones_w,
                                      (((1,), (1,)), ((), ())),
                                      preferred_element_type=_F32)
        kn = jnp.sqrt(jnp.sum(wk * wk, axis=1, keepdims=True))
        dot = jnp.transpose(dot_col)
        mn2_row = jnp.transpose(mn2_col)
        sim = dot / jnp.maximum(kn * jnp.sqrt(mn2_row), 1e-8)
        e = jnp.exp(sim - jnp.max(sim, axis=1, keepdims=True))
        wc = e / jnp.sum(e, axis=1, keepdims=True) * wstr_ref[0]

        ww = (ag_ref[0] * (alloc - wc) + wc) * wg_ref[0]
        wwall_s[pl.ds(ks, 1), :] = ww
        ww_col = jnp.transpose(ww)

        memnew = mem * (1.0 - ww_col * evec_ref[0]) + ww_col * wvec_ref[0]
        memnew_ref[0] = memnew
        memall_s[pl.ds(ks * _N, _N), :] = memnew

        rk = rkeys_ref[0]
        dot_r = jax.lax.dot_general(rk, memnew, (((1,), (1,)), ((), ())),
                                    preferred_element_type=_F32)
        mn2b = jax.lax.dot_general(ones_w, memnew * memnew,
                                   (((1,), (1,)), ((), ())),
                                   preferred_element_type=_F32)
        rn = jnp.sqrt(jnp.sum(rk * rk, axis=1, keepdims=True))
        sim_r = dot_r / jnp.maximum(rn * jnp.sqrt(mn2b), 1e-8)
        e_r = jnp.exp(sim_r - jnp.max(sim_r, axis=1, keepdims=True))
        cwall_s[:, pl.ds(ks * _N, _N)] = (
            e_r / jnp.sum(e_r, axis=1, keepdims=True) * rstr_ref[0])

    @pl.when(i == 0)
    def _():
        wwcol_s[...] = jnp.transpose(wwall_s[pl.ds(b, 1), :])
        bw_s[...] = jnp.zeros((_R, _N), _F32)

    ww = wwall_s[pl.ds(b, 1), :]
    prec = prec_ref[0]
    idx = pl.multiple_of(i * _TI, _TI)
    wwi = wwcol_s[pl.ds(idx, _TI), :]
    L = L_ref[0]

    Lnew = L * (1.0 - wwi - ww) + wwi * prec
    Lout_ref[0] = Lnew

    rw = rw_ref[0]
    fw_strip = jax.lax.dot_general(rw, Lnew, (((1,), (1,)), ((), ())),
                                   preferred_element_type=_F32)
    fw_s[:, pl.ds(idx, _TI)] = fw_strip

    bw_s[...] += jax.lax.dot_general(rws_ref[0], Lnew, (((1,), (0,)), ((), ())),
                                     preferred_element_type=_F32)

    @pl.when(i == _NI - 1)
    def _():
        rm = rm_ref[0]
        cw = cwall_s[:, pl.ds(b * _N, _N)]
        rw_new = (fw_s[...] * rm[:, 0:1] + bw_s[...] * rm[:, 1:2]
                  + cw * rm[:, 2:3])
        rwout_ref[0] = rw_new

        rv_ref[0] = jax.lax.dot_general(
            rw_new, memall_s[pl.ds(b * _N, _N), :], (((1,), (0,)), ((), ())),
            preferred_element_type=_F32)

        prodw = (rw_new[0:1] * rw_new[1:2]) * (rw_new[2:3] * rw_new[3:4])
        ret = 1.0 - fg_ref[0] * prodw
        uold = u2_ref[0]
        uout_ref[0] = (uold + ww - uold * ww) * ret

    @pl.when(jnp.logical_and(b == _B - 1, i == _NI - 1))
    def _():
        wsum = jnp.sum(wwall_s[...])
        prec_all = prec_all_ref[...]
        pout_ref[:, 0, :] = (1.0 - wsum) * prec_all[:, 0, :] + wwall_s[...]


def _kidx(b, i):
    return jnp.minimum(b * _NI + i, _B - 1)


def kernel(memory, usage, read_weights, temporal_link, precedence, write_key,
           write_strength, allocation_gate, write_gate, write_vector,
           erase_vector, read_keys, read_strength, read_modes, free_gates):
    f32 = _F32
    usage_t = jnp.transpose(usage, (0, 2, 1))

    (L_new, mem_new, rw_new, read_val, usage_new, prec_new) = pl.pallas_call(
        _body,
        grid=(_B, _NI),
        in_specs=[
            pl.BlockSpec((1, _N, _W), lambda b, i: (_kidx(b, i), 0, 0)),
            pl.BlockSpec((1, 1, _N), lambda b, i: (_kidx(b, i), 0, 0)),
            pl.BlockSpec((1, _N, 1), lambda b, i: (_kidx(b, i), 0, 0)),
            pl.BlockSpec((1, 1, _W), lambda b, i: (_kidx(b, i), 0, 0)),
            pl.BlockSpec((1, 1, 1), lambda b, i: (_kidx(b, i), 0, 0)),
            pl.BlockSpec((1, 1, 1), lambda b, i: (_kidx(b, i), 0, 0)),
            pl.BlockSpec((1, 1, 1), lambda b, i: (_kidx(b, i), 0, 0)),
            pl.BlockSpec((1, 1, _W), lambda b, i: (_kidx(b, i), 0, 0)),
            pl.BlockSpec((1, 1, _W), lambda b, i: (_kidx(b, i), 0, 0)),
            pl.BlockSpec((1, _R, _W), lambda b, i: (_kidx(b, i), 0, 0)),
            pl.BlockSpec((1, _R, 1), lambda b, i: (_kidx(b, i), 0, 0)),
            pl.BlockSpec((1, _TI, _N), lambda b, i: (b, i, 0)),
            pl.BlockSpec((1, _R, _TI), lambda b, i: (b, 0, i)),
            pl.BlockSpec((1, 1, _N), lambda b, i: (b, 0, 0)),
            pl.BlockSpec((1, _R, _N), lambda b, i: (b, 0, 0)),
            pl.BlockSpec((1, _R, 3), lambda b, i: (b, 0, 0)),
            pl.BlockSpec((1, 1, _N), lambda b, i: (b, 0, 0)),
            pl.BlockSpec((1, 1, _N), lambda b, i: (b, 0, 0)),
            pl.BlockSpec((_B, 1, _N), lambda b, i: (0, 0, 0)),
        ],
        out_specs=[
            pl.BlockSpec((1, _TI, _N), lambda b, i: (b, i, 0)),
            pl.BlockSpec((1, _N, _W), lambda b, i: (_kidx(b, i), 0, 0)),
            pl.BlockSpec((1, _R, _N), lambda b, i: (b, 0, 0)),
            pl.BlockSpec((1, _R, _W), lambda b, i: (b, 0, 0)),
            pl.BlockSpec((1, 1, _N), lambda b, i: (b, 0, 0)),
            pl.BlockSpec((_B, 1, _N), lambda b, i: (0, 0, 0)),
        ],
        out_shape=[
            jax.ShapeDtypeStruct((_B, _N, _N), f32),
            jax.ShapeDtypeStruct((_B, _N, _W), f32),
            jax.ShapeDtypeStruct((_B, _R, _N), f32),
            jax.ShapeDtypeStruct((_B, _R, _W), f32),
            jax.ShapeDtypeStruct((_B, 1, _N), f32),
            jax.ShapeDtypeStruct((_B, 1, _N), f32),
        ],
        scratch_shapes=[
            pltpu.VMEM((_B, _N), f32),
            pltpu.VMEM((_N, 1), f32),
            pltpu.VMEM((_R, _B * _N), f32),
            pltpu.VMEM((_B * _N, _W), f32),
            pltpu.VMEM((_R, _N), f32),
            pltpu.VMEM((_R, _N), f32),
        ],
    )(memory, usage, usage_t, write_key, write_strength, allocation_gate,
      write_gate, write_vector, erase_vector, read_keys, read_strength,
      temporal_link, read_weights,
      precedence, read_weights, read_modes, free_gates, usage, precedence)

    return (read_val, mem_new, usage_new, rw_new, L_new, prec_new)
